# baseline (speedup 1.0000x reference)
import functools

import jax
import jax.numpy as jnp
from jax import lax
from jax.experimental import pallas as pl
from jax.experimental.pallas import tpu as pltpu
from jax.experimental.pallas import tpu_sc as plsc

VOCAB = 1000000
EMB_DIM = 64
BATCH = 4096
HIST = 50
N = BATCH * HIST
NW = 32
G = 128
NB = VOCAB // G
NBT = NB + 1
SROWS = NBT * (G // 2)
MAXB = (NBT + NW - 1) // NW
NGRP = N // G
K = 8
UNITS = NGRP // K
MAXU = (UNITS + NW - 1) // NW

_CP = pltpu.CompilerParams(use_tc_tiling_on_sc=True, needs_layout_passes=False)
_MESH = plsc.VectorSubcoreMesh(core_axis_name="c", subcore_axis_name="s")


@functools.partial(
    pl.kernel,
    mesh=_MESH,
    out_type=jax.ShapeDtypeStruct((SROWS, G), jnp.float32),
    scratch_types=[
        pltpu.VMEM((64, G + 1), jnp.float32),
        pltpu.VMEM((64, G + 1), jnp.float32),
        pltpu.VMEM((64, G), jnp.float32),
        pltpu.VMEM((64, G), jnp.float32),
        pltpu.SemaphoreType.DMA,
        pltpu.SemaphoreType.DMA,
        pltpu.SemaphoreType.DMA,
        pltpu.SemaphoreType.DMA,
    ],
    compiler_params=_CP,
)
def _sc_detrans(tab_t, tailb, scratch_hbm, b0, b1, t0, t1, g0, g1, o0, o1):
    wid = lax.axis_index("s") * 2 + lax.axis_index("c")
    bb = (b0, b1)
    tb = (t0, t1)
    gsem = (g0, g1)
    osem = (o0, o1)
    iota = lax.iota(jnp.int32, 16)
    nblk = lax.select(wid < NBT - (MAXB - 1) * NW, MAXB, MAXB - 1)

    def fire_fetch(j, slot):
        blk = wid + j * NW

        @pl.when(blk < NB)
        def _():
            pltpu.async_copy(
                tab_t.at[:, pl.ds(pl.multiple_of(blk * G, G), G)],
                bb[slot].at[:, pl.ds(0, G)],
                gsem[slot],
            )

        @pl.when(blk == NB)
        def _():
            pltpu.async_copy(tailb, bb[slot].at[:, pl.ds(0, G)], gsem[slot])

    def wait_fetch(slot):
        pltpu.make_async_copy(
            tailb, bb[slot].at[:, pl.ds(0, G)], gsem[slot]
        ).wait()

    def wait_flush(slot):
        pltpu.make_async_copy(
            tb[slot], scratch_hbm.at[pl.ds(0, 64)], osem[slot]
        ).wait()

    def transpose(slot):
        @plsc.parallel_loop(0, 8, unroll=2)
        def _(c0):
            jvec = c0 * 16 + iota
            qvec = jvec >> 1
            colbase = (jvec & 1) * 64

            @plsc.parallel_loop(0, 8, unroll=2)
            def _(d0):
                for dd in range(8):
                    d = d0 * 8 + dd
                    val = bb[slot][d, pl.ds(c0 * 16, 16)]
                    plsc.store_scatter(tb[slot], [qvec, colbase + d], val)

    def stage(j, slot):
        @pl.when(j < nblk)
        def _():
            blk = wid + j * NW
            wait_fetch(slot)

            @pl.when(j >= 2)
            def _():
                wait_flush(slot)

            transpose(slot)
            pltpu.async_copy(
                tb[slot],
                scratch_hbm.at[pl.ds(pl.multiple_of(blk * 64, 8), 64)],
                osem[slot],
            )
            fire_fetch(j + 2, slot)

    fire_fetch(0, 0)
    fire_fetch(1, 1)

    def body(i, carry):
        stage(2 * i, 0)
        stage(2 * i + 1, 1)
        return carry

    lax.fori_loop(0, (MAXB + 1) // 2, body, 0)
    wait_flush(0)
    wait_flush(1)


@functools.partial(
    pl.kernel,
    mesh=_MESH,
    out_type=jax.ShapeDtypeStruct((HIST, EMB_DIM, BATCH), jnp.float32),
    scratch_types=[
        pltpu.VMEM((HIST, G), jnp.int32),
        pltpu.VMEM((HIST, G), jnp.int32),
        pltpu.VMEM((G, G + 1), jnp.float32),
        pltpu.VMEM((G, G + 1), jnp.float32),
        pltpu.VMEM((64, G), jnp.float32),
        pltpu.VMEM((64, G), jnp.float32),
        pltpu.SemaphoreType.DMA,
        pltpu.SemaphoreType.DMA,
        pltpu.SemaphoreType.DMA,
        pltpu.SemaphoreType.DMA,
    ],
    compiler_params=_CP,
)
def _sc_gather(idx_hbm, scratch_hbm, out_hbm, idx_v, pidx_v,
               rb0, rb1, ob0, ob1, g0, g1, o0, o1):
    wid = lax.axis_index("s") * 2 + lax.axis_index("c")
    rb = (rb0, rb1)
    ob = (ob0, ob1)
    gsem = (g0, g1)
    osem = (o0, o1)
    iota = lax.iota(jnp.int32, 16)

    pltpu.sync_copy(idx_hbm.at[wid], idx_v)

    @plsc.parallel_loop(0, HIST, unroll=2)
    def _(i):
        for ci in range(8):
            sl = pl.ds(ci * 16, 16)
            pidx_v[i, sl] = idx_v[i, sl] >> 1

    def fire(t, slot):
        @pl.when(t < HIST)
        def _():
            pltpu.async_copy(
                scratch_hbm.at[pidx_v.at[t]],
                rb[slot].at[:, pl.ds(0, G)],
                gsem[slot],
            )

    def wait_fetch(slot):
        pltpu.make_async_copy(
            scratch_hbm.at[pl.ds(0, G)],
            rb[slot].at[:, pl.ds(0, G)],
            gsem[slot],
        ).wait()

    def wait_out(slot):
        pltpu.make_async_copy(
            ob[slot], out_hbm.at[0].at[:, pl.ds(0, G)], osem[slot]
        ).wait()

    def extract(t, slot):
        @plsc.parallel_loop(0, 8, unroll=2)
        def _(c0):
            lvec = c0 * 16 + iota
            ivec = plsc.load_gather(
                idx_v, [jnp.zeros((16,), jnp.int32) + t, lvec]
            )
            scol = (ivec & 1) * 64

            @plsc.parallel_loop(0, 8, unroll=2)
            def _(d0):
                for dd in range(8):
                    d = d0 * 8 + dd
                    val = plsc.load_gather(rb[slot], [lvec, scol + d])
                    ob[slot][d, pl.ds(c0 * 16, 16)] = val

    def stage(t, slot):
        @pl.when(t < HIST)
        def _():
            wait_fetch(slot)

            @pl.when(t >= 2)
            def _():
                wait_out(slot)

            extract(t, slot)
            pltpu.async_copy(
                ob[slot],
                out_hbm.at[t].at[
                    :, pl.ds(pl.multiple_of(wid * G, G), G)
                ],
                osem[slot],
            )
            fire(t + 2, slot)

    fire(0, 0)
    fire(1, 1)

    def body(i, carry):
        stage(2 * i, 0)
        stage(2 * i + 1, 1)
        return carry

    lax.fori_loop(0, HIST // 2, body, 0)
    wait_out(0)
    wait_out(1)


def kernel(batch, table):
    table_t = table.T
    tailb = jnp.zeros((EMB_DIM, G), jnp.float32)
    tailb = tailb.at[:, : VOCAB - NB * G].set(table_t[:, NB * G :])
    idx3 = (
        batch.T.astype(jnp.int32)
        .reshape(HIST, BATCH // G, G)
        .transpose(1, 0, 2)
    )
    scratch = _sc_detrans(table_t, tailb)
    out = _sc_gather(idx3, scratch)
    return out.transpose(2, 0, 1)


# speedup vs baseline: 1.3775x; 1.3775x over previous
import functools

import jax
import jax.numpy as jnp
from jax import lax
from jax.experimental import pallas as pl
from jax.experimental.pallas import tpu as pltpu
from jax.experimental.pallas import tpu_sc as plsc

VOCAB = 1000000
EMB_DIM = 64
BATCH = 4096
HIST = 50
N = BATCH * HIST
NW = 32
G = 128
GPW = N // NW // G
K = 5
UPW = GPW // K
UNITS = NW * UPW


@functools.partial(
    pl.kernel,
    mesh=plsc.VectorSubcoreMesh(core_axis_name="c", subcore_axis_name="s"),
    out_type=jax.ShapeDtypeStruct((UNITS, K, G, EMB_DIM), jnp.float32),
    scratch_types=[
        pltpu.VMEM((GPW, G), jnp.int32),
        pltpu.VMEM((K, G, EMB_DIM), jnp.float32),
        pltpu.VMEM((K, G, EMB_DIM), jnp.float32),
        pltpu.SemaphoreType.DMA,
        pltpu.SemaphoreType.DMA,
        pltpu.SemaphoreType.DMA,
        pltpu.SemaphoreType.DMA,
    ],
    compiler_params=pltpu.CompilerParams(use_tc_tiling_on_sc=False),
)
def _sc_gather(idx_hbm, table_hbm, out_hbm, idx_v, rows0, rows1,
               g0, g1, o0, o1):
    wid = lax.axis_index("s") * 2 + lax.axis_index("c")
    rows = (rows0, rows1)
    gsem = (g0, g1)
    osem = (o0, o1)

    pltpu.sync_copy(idx_hbm.at[wid], idx_v)

    gathers = [None, None]
    outs = [None, None]

    def fire_unit(i):
        b = i % 2
        cps = []
        for j in range(K):
            cps.append(
                pltpu.async_copy(
                    table_hbm.at[idx_v.at[i * K + j]], rows[b].at[j], gsem[b]
                )
            )
        gathers[b] = cps

    def retire_unit(i):
        b = i % 2
        for cp in gathers[b]:
            cp.wait()
        gathers[b] = None
        outs[b] = pltpu.async_copy(
            rows[b], out_hbm.at[wid * UPW + i], osem[b]
        )

    for i in range(UPW):
        b = i % 2
        if outs[b] is not None:
            outs[b].wait()
            outs[b] = None
        fire_unit(i)
        if i >= 1:
            retire_unit(i - 1)
    retire_unit(UPW - 1)
    for b in range(2):
        if outs[b] is not None:
            outs[b].wait()


def kernel(batch, table):
    idx = batch.reshape(NW, GPW, G).astype(jnp.int32)
    out = _sc_gather(idx, table)
    return out.reshape(BATCH, HIST, EMB_DIM)
